# baseline (speedup 1.0000x reference)
import jax
import jax.numpy as jnp
from jax.experimental import pallas as pl


def _body(x0, x1, x2, x3, o_ref):
    i = pl.program_id(0)
    acc = x0[0, 0, :] + x1[0, 0, :] + x2[0, 0, :] + x3[0, 0, :]
    o_ref[pl.ds(i // 4, 1), :] = acc.reshape(1, -1)


def kernel(inputs):
    B, S, D = inputs.shape

    def spec(q):
        return pl.BlockSpec(
            (1, 256, D), lambda i, q=q: (i // 4, (i % 4) * 4 + q, 0)
        )

    return pl.pallas_call(
        _body,
        grid=(B * 4,),
        in_specs=[spec(q) for q in range(4)],
        out_specs=pl.BlockSpec((B, D), lambda i: (0, 0)),
        out_shape=jax.ShapeDtypeStruct((B, D), inputs.dtype),
    )(*([inputs] * 4))


# speedup vs baseline: 1.0763x; 1.0763x over previous
import jax
import jax.numpy as jnp
from jax.experimental import pallas as pl
from jax.experimental.pallas import tpu as pltpu

NSTREAM = 4
_MXU_N = 128
_SQ = 512
_HALF = NSTREAM * _SQ


def _count(x, ones_j):
    sq, d = x.shape
    chunks = [jnp.abs(x[:, c * 128:(c + 1) * 128]) for c in range(d // 128)]
    while len(chunks) > 1:
        chunks = [
            jnp.maximum(chunks[i], chunks[i + 1])
            if i + 1 < len(chunks) else chunks[i]
            for i in range(0, len(chunks), 2)
        ]
    rs = jax.lax.dot_general(
        chunks[0].astype(jnp.bfloat16), ones_j, (((1,), (0,)), ((), ())),
        preferred_element_type=jnp.float32,
    )
    return jnp.sum(jnp.minimum(rs, 1.0))


def _body(x0, x1, x2, x3, o_ref, stash_ref, cnt_ref):
    refs = (x0, x1, x2, x3)
    i = pl.program_id(0)
    b = i // 2
    h = i % 2
    ones_j = jnp.ones((128, _MXU_N), dtype=jnp.bfloat16)
    cnt_h = (
        _count(x0[0], ones_j) + _count(x1[0], ones_j)
        + _count(x2[0], ones_j) + _count(x3[0], ones_j)
    ) * (1.0 / _MXU_N)

    @pl.when(h == 0)
    def _first_half():
        cnt_ref[0] = cnt_h
        for q in range(NSTREAM):
            stash_ref[q * _SQ:(q + 1) * _SQ, :] = refs[q][0]

    @pl.when(h == 1)
    def _second_half():
        cnt_f = cnt_ref[0] + cnt_h
        idx = jnp.maximum(cnt_f - 1.0, 0.0).astype(jnp.int32)
        in_first = idx < _HALF
        idx0 = jnp.where(in_first, idx, 0)
        row0 = stash_ref[pl.ds(idx0, 1), :]
        rel = jnp.where(in_first, 0, idx - _HALF)
        q = rel // _SQ
        off = rel % _SQ
        row1 = refs[NSTREAM - 1][0, pl.ds(off, 1), :]
        for j in range(NSTREAM - 2, -1, -1):
            row1 = jnp.where(q == j, refs[j][0, pl.ds(off, 1), :], row1)
        o_ref[pl.ds(b, 1), :] = jnp.where(in_first, row0, row1)


def kernel(inputs):
    B, S, D = inputs.shape

    def spec(q):
        return pl.BlockSpec(
            (1, _SQ, D), lambda i, q=q: (i // 2, (i % 2) * NSTREAM + q, 0)
        )

    return pl.pallas_call(
        _body,
        grid=(B * 2,),
        in_specs=[spec(q) for q in range(NSTREAM)],
        out_specs=pl.BlockSpec((B, D), lambda i: (0, 0)),
        out_shape=jax.ShapeDtypeStruct((B, D), inputs.dtype),
        scratch_shapes=[
            pltpu.VMEM((_HALF, D), jnp.float32),
            pltpu.SMEM((1,), jnp.float32),
        ],
    )(*([inputs] * NSTREAM))


# speedup vs baseline: 1.0997x; 1.0217x over previous
import jax
import jax.numpy as jnp
from jax.experimental import pallas as pl
from jax.experimental.pallas import tpu as pltpu

NSTREAM = 4
_MXU_N = 128
_SQ = 512
_HALF = NSTREAM * _SQ


def _count(x, ones_j):
    sq, d = x.shape
    chunks = [jnp.abs(x[:, c * 128:(c + 1) * 128]) for c in range(d // 128)]
    while len(chunks) > 1:
        chunks = [
            jnp.maximum(chunks[i], chunks[i + 1])
            if i + 1 < len(chunks) else chunks[i]
            for i in range(0, len(chunks), 2)
        ]
    rs = jax.lax.dot_general(
        chunks[0].astype(jnp.bfloat16), ones_j, (((1,), (0,)), ((), ())),
        preferred_element_type=jnp.float32,
    )
    return jnp.sum(jnp.minimum(rs, 1.0))


def _body(x0, x1, x2, x3, hbm_ref, o_ref, row_buf, cnt_ref, sem):
    refs = (x0, x1, x2, x3)
    i = pl.program_id(0)
    b = i // 2
    h = i % 2
    ones_j = jnp.ones((128, _MXU_N), dtype=jnp.bfloat16)
    cnt_h = (
        _count(x0[0], ones_j) + _count(x1[0], ones_j)
        + _count(x2[0], ones_j) + _count(x3[0], ones_j)
    ) * (1.0 / _MXU_N)

    @pl.when(h == 0)
    def _first_half():
        cnt_ref[0] = cnt_h

    @pl.when(h == 1)
    def _second_half():
        cnt_f = cnt_ref[0] + cnt_h
        idx = jnp.maximum(cnt_f - 1.0, 0.0).astype(jnp.int32)
        in_first = idx < _HALF

        rel = jnp.where(in_first, 0, idx - _HALF)
        q = rel // _SQ
        off = rel % _SQ
        row = refs[NSTREAM - 1][0, pl.ds(off, 1), :]
        for j in range(NSTREAM - 2, -1, -1):
            row = jnp.where(q == j, refs[j][0, pl.ds(off, 1), :], row)
        o_ref[pl.ds(b, 1), :] = row

        @pl.when(in_first)
        def _fetch_first_half_row():
            cp = pltpu.make_async_copy(
                hbm_ref.at[b, pl.ds(idx, 1), :], row_buf, sem
            )
            cp.start()
            cp.wait()
            o_ref[pl.ds(b, 1), :] = row_buf[:, :]


def kernel(inputs):
    B, S, D = inputs.shape

    def spec(q):
        return pl.BlockSpec(
            (1, _SQ, D), lambda i, q=q: (i // 2, (i % 2) * NSTREAM + q, 0)
        )

    return pl.pallas_call(
        _body,
        grid=(B * 2,),
        in_specs=[spec(q) for q in range(NSTREAM)]
        + [pl.BlockSpec(memory_space=pl.ANY)],
        out_specs=pl.BlockSpec((B, D), lambda i: (0, 0)),
        out_shape=jax.ShapeDtypeStruct((B, D), inputs.dtype),
        scratch_shapes=[
            pltpu.VMEM((1, D), jnp.float32),
            pltpu.SMEM((1,), jnp.float32),
            pltpu.SemaphoreType.DMA,
        ],
    )(*([inputs] * (NSTREAM + 1)))
